# baseline (speedup 1.0000x reference)
import functools

import jax
import jax.numpy as jnp
from jax import lax
from jax.experimental import pallas as pl
from jax.experimental.pallas import tpu as pltpu
from jax.experimental.pallas import tpu_sc as plsc

NC, NS, LANES = 2, 16, 16
NW = NC * NS

B, L = 4096, 50
D, R = 32, 64
BPW = B // NW
NBUF = 4
GRP = 4


RCHUNK = 8192
V = 1000001
NSTEP = -(-V // RCHUNK)
NROWS = NSTEP * RCHUNK


def _repack_body(x_ref, o_ref):
    eye = (jax.lax.broadcasted_iota(jnp.int32, (D, D), 0)
           == jax.lax.broadcasted_iota(jnp.int32, (D, D), 1)).astype(jnp.float32)
    q = RCHUNK // 4
    for a in range(4):
        xa = x_ref[:, a * q:(a + 1) * q]
        ya = jax.lax.dot_general(xa, eye, (((0,), (0,)), ((), ())),
                                 preferred_element_type=jnp.float32)
        o_ref[:, a * D:(a + 1) * D] = ya


_repack = pl.pallas_call(
    _repack_body,
    grid=(NSTEP,),
    in_specs=[pl.BlockSpec((D, RCHUNK), lambda g: (0, g))],
    out_specs=pl.BlockSpec((RCHUNK // 4, 128), lambda g: (g, 0)),
    out_shape=jax.ShapeDtypeStruct((NROWS // 4, 128), jnp.float32),
)


_DNUMS = jax.lax.GatherDimensionNumbers(
    offset_dims=(), collapsed_slice_dims=(0,), start_index_map=(0,)
)


def _bcast_lane(vec, d):
    idx = jnp.full((LANES, 1), d, jnp.int32)
    return jax.lax.gather(
        vec, idx, _DNUMS, (1,),
        mode=jax.lax.GatherScatterMode.PROMISE_IN_BOUNDS,
    )


def _lookup_body(inputs_hbm, table_hbm, w_hbm, b_hbm, out_hbm,
                 idx_v, rows_v, w_v, b_v, out_v, sems):
    wid = lax.axis_index("s") * NC + lax.axis_index("c")
    base = wid * BPW
    pltpu.sync_copy(inputs_hbm.at[pl.ds(base, BPW)], idx_v)
    pltpu.sync_copy(w_hbm, w_v)
    pltpu.sync_copy(b_hbm, b_v)

    for j in range(NBUF):
        pltpu.async_copy(table_hbm.at[idx_v.at[j]], rows_v.at[j], sems.at[j])

    def group(g, carry):
        for u in range(GRP):
            i = g * GRP + u
            j = i % NBUF
            pltpu.make_async_copy(
                table_hbm.at[idx_v.at[i]], rows_v.at[j], sems.at[j]
            ).wait()
            buf = rows_v.at[j]
            acc0 = buf[0, 0:16]
            acc1 = buf[0, 16:32]
            for l in range(1, L):
                acc0 = acc0 + buf[l, 0:16]
                acc1 = acc1 + buf[l, 16:32]

            @pl.when(i < BPW - NBUF)
            def _():
                pltpu.async_copy(
                    table_hbm.at[idx_v.at[i + NBUF]], rows_v.at[j], sems.at[j]
                )

            o0 = b_v[0:16]
            o1 = b_v[16:32]
            o2 = b_v[32:48]
            o3 = b_v[48:64]
            for d in range(D):
                s_d = _bcast_lane(acc0 if d < LANES else acc1, d % LANES)
                o0 = o0 + s_d * w_v[d, 0:16]
                o1 = o1 + s_d * w_v[d, 16:32]
                o2 = o2 + s_d * w_v[d, 32:48]
                o3 = o3 + s_d * w_v[d, 48:64]
            out_v[i, 0:16] = o0
            out_v[i, 16:32] = o1
            out_v[i, 32:48] = o2
            out_v[i, 48:64] = o3
        return carry

    lax.fori_loop(0, BPW // GRP, group, 0)
    pltpu.sync_copy(out_v, out_hbm.at[pl.ds(base, BPW)])


_lookup = functools.partial(
    pl.kernel,
    out_type=jax.ShapeDtypeStruct((B, R), jnp.float32),
    mesh=plsc.VectorSubcoreMesh(core_axis_name="c", subcore_axis_name="s"),
    scratch_types=[
        pltpu.VMEM((BPW, L), jnp.int32),
        pltpu.VMEM((NBUF, L, D), jnp.float32),
        pltpu.VMEM((D, R), jnp.float32),
        pltpu.VMEM((R,), jnp.float32),
        pltpu.VMEM((BPW, R), jnp.float32),
        pltpu.SemaphoreType.DMA((NBUF,)),
    ],
    compiler_params=pltpu.CompilerParams(use_tc_tiling_on_sc=False),
)(_lookup_body)


def kernel(inputs, emb_table, W, b):
    t32 = emb_table.T
    tp = _repack(t32)
    tpv = tp.reshape(NROWS, D)
    inputs_p = (inputs & ~8191) | ((inputs & 2047) << 2) | ((inputs >> 11) & 3)
    return _lookup(inputs_p, tpv, W, b)
